# baseline (speedup 1.0000x reference)
import functools

import jax
import jax.numpy as jnp
from jax import lax
from jax.experimental import pallas as pl
from jax.experimental.pallas import tpu as pltpu
from jax.experimental.pallas import tpu_sc as plsc

IDXW = 128
CH = 4
CHUNK = CH * IDXW
NBUF = 2
LANES = 16


def _build(B, L, V, D, NC, NS):
    NW = NC * NS
    ROWS = B * L
    rows_w = ROWS // NW
    n_chunks = rows_w // CHUNK
    idx_rows_w = rows_w // IDXW

    mesh = plsc.VectorSubcoreMesh(
        core_axis_name="c", subcore_axis_name="s",
        num_cores=NC, num_subcores=NS)

    @functools.partial(
        pl.kernel,
        out_type=jax.ShapeDtypeStruct((ROWS, D), jnp.float32),
        mesh=mesh,
        scratch_types=[
            pltpu.VMEM((idx_rows_w, IDXW), jnp.int32),
            pltpu.VMEM((L, D), jnp.float32),
            pltpu.VMEM((NBUF, CHUNK, D), jnp.float32),
            pltpu.SemaphoreType.DMA,
            pltpu.SemaphoreType.DMA,
            pltpu.SemaphoreType.DMA,
        ],
        compiler_params=pltpu.CompilerParams(use_tc_tiling_on_sc=False),
    )
    def body(idx_hbm, table_hbm, pos_hbm, out_hbm,
             idx_v, pos_v, rows_v, gsem0, gsem1, osem):
        gsems = (gsem0, gsem1)
        wid = lax.axis_index("s") * NC + lax.axis_index("c")
        idx_base = wid * idx_rows_w
        row_base = wid * rows_w

        pltpu.sync_copy(pos_hbm, pos_v)
        pltpu.sync_copy(idx_hbm.at[pl.ds(idx_base, idx_rows_w)], idx_v)

        def fire_gather(c, b):
            for j in range(CH):
                pltpu.async_copy(
                    table_hbm.at[idx_v.at[c * CH + j]],
                    rows_v.at[b, pl.ds(j * IDXW, IDXW)],
                    gsems[b])

        def drain_gather(b):
            pltpu.make_async_copy(
                table_hbm.at[pl.ds(0, CHUNK)], rows_v.at[b], gsems[b]).wait()

        fire_gather(0, 0)
        fire_gather(1, 1)

        @pl.loop(0, n_chunks, step=NBUF)
        def _outer(t):
            for b in range(NBUF):
                c = t + b
                drain_gather(b)
                p0 = lax.rem(c * CHUNK, L)

                @pl.loop(0, CHUNK, init_carry=p0, unroll=4)
                def _add(r, p):
                    for q in range(D // LANES):
                        plsc.addupdate(
                            rows_v.at[b, r, pl.ds(q * LANES, LANES)],
                            pos_v[p, pl.ds(q * LANES, LANES)])
                    return jnp.where(p + 1 == L, 0, p + 1)

                st = pltpu.async_copy(
                    rows_v.at[b],
                    out_hbm.at[pl.ds(row_base + c * CHUNK, CHUNK)],
                    osem)
                st.wait()

                @pl.when(c + NBUF < n_chunks)
                def _():
                    fire_gather(c + NBUF, b)

    return body


def kernel(inputs, token_table, pos_table):
    B, L = inputs.shape
    V, D = token_table.shape
    info = plsc.get_sparse_core_info()
    NC, NS = info.num_cores, info.num_subcores
    idx2d = inputs.astype(jnp.int32).reshape((B * L) // IDXW, IDXW)
    out = _build(B, L, V, D, NC, NS)(idx2d, token_table, pos_table)
    return out.reshape(B, L, D)


# speedup vs baseline: 1.2207x; 1.2207x over previous
import functools

import jax
import jax.numpy as jnp
from jax import lax
from jax.experimental import pallas as pl
from jax.experimental.pallas import tpu as pltpu
from jax.experimental.pallas import tpu_sc as plsc

IDXW = 128
SEQ_PER_CHUNK = 2
NBUF = 3
LANES = 16


def _build(B, L, V, D, NC, NS):
    NW = NC * NS
    seqs_w = B // NW
    n_chunks = seqs_w // SEQ_PER_CHUNK
    rem = L - IDXW

    mesh = plsc.VectorSubcoreMesh(
        core_axis_name="c", subcore_axis_name="s",
        num_cores=NC, num_subcores=NS)

    @functools.partial(
        pl.kernel,
        out_type=jax.ShapeDtypeStruct((B, L, D), jnp.float32),
        mesh=mesh,
        scratch_types=[
            pltpu.VMEM((seqs_w, L), jnp.int32),
            pltpu.VMEM((L, D), jnp.float32),
            pltpu.VMEM((NBUF, SEQ_PER_CHUNK, L, D), jnp.float32),
            pltpu.SemaphoreType.DMA,
            pltpu.SemaphoreType.DMA,
            pltpu.SemaphoreType.DMA,
            pltpu.SemaphoreType.DMA,
            pltpu.SemaphoreType.DMA,
            pltpu.SemaphoreType.DMA,
        ],
        compiler_params=pltpu.CompilerParams(use_tc_tiling_on_sc=False),
    )
    def body(idx_hbm, table_hbm, pos_hbm, out_hbm,
             idx_v, pos_v, rows_v, g0, g1, g2, o0, o1, o2):
        gsems = (g0, g1, g2)
        osems = (o0, o1, o2)
        wid = lax.axis_index("s") * NC + lax.axis_index("c")
        seq_base = wid * seqs_w

        pltpu.sync_copy(pos_hbm, pos_v)
        pltpu.sync_copy(idx_hbm.at[pl.ds(seq_base, seqs_w)], idx_v)

        def fire_gather(c, b):
            for s in range(SEQ_PER_CHUNK):
                row = c * SEQ_PER_CHUNK + s
                pltpu.async_copy(
                    table_hbm.at[idx_v.at[row, pl.ds(0, IDXW)]],
                    rows_v.at[b, s, pl.ds(0, IDXW)],
                    gsems[b])
                pltpu.async_copy(
                    table_hbm.at[idx_v.at[row, pl.ds(IDXW, rem)]],
                    rows_v.at[b, s, pl.ds(IDXW, rem)],
                    gsems[b])

        def drain_gather(b):
            pltpu.make_async_copy(
                table_hbm.at[pl.ds(0, L)], rows_v.at[b, 0], gsems[b]).wait()
            pltpu.make_async_copy(
                table_hbm.at[pl.ds(0, L)], rows_v.at[b, 1], gsems[b]).wait()

        def drain_store(b):
            pltpu.make_async_copy(
                rows_v.at[b], out_hbm.at[pl.ds(0, SEQ_PER_CHUNK)],
                osems[b]).wait()

        def trip(c, b):
            drain_gather(b)
            for s in range(SEQ_PER_CHUNK):
                @pl.loop(0, L, unroll=4)
                def _add(r):
                    for q in range(D // LANES):
                        plsc.addupdate(
                            rows_v.at[b, s, r, pl.ds(q * LANES, LANES)],
                            pos_v[r, pl.ds(q * LANES, LANES)])
            pltpu.async_copy(
                rows_v.at[b],
                out_hbm.at[pl.ds(seq_base + c * SEQ_PER_CHUNK, SEQ_PER_CHUNK)],
                osems[b])
            bn = (b + 2) % NBUF

            @pl.when(jnp.logical_and(c >= 1, c + 2 < n_chunks))
            def _():
                drain_store(bn)

            @pl.when(c + 2 < n_chunks)
            def _():
                fire_gather(c + 2, bn)

        fire_gather(0, 0)
        fire_gather(1, 1)

        n_main = (n_chunks // NBUF) * NBUF

        @pl.loop(0, n_main, step=NBUF)
        def _outer(t):
            for db in range(NBUF):
                trip(t + db, db)

        for c in range(n_main, n_chunks):
            trip(c, c % NBUF)

        for c in range(n_chunks - NBUF, n_chunks):
            drain_store(c % NBUF)

    return body


def kernel(inputs, token_table, pos_table):
    B, L = inputs.shape
    V, D = token_table.shape
    info = plsc.get_sparse_core_info()
    NC, NS = info.num_cores, info.num_subcores
    out = _build(B, L, V, D, NC, NS)(
        inputs.astype(jnp.int32), token_table, pos_table)
    return out
